# baseline (device time: 37891 ns/iter reference)
import jax
import jax.numpy as jnp
from jax import lax
from jax.experimental import pallas as pl
from jax.experimental.pallas import tpu as pltpu

N_DEV = 16
EPS = 1e-5
N_BLK = 16
N_BLK_B = 8


def _allreduce_inv(x):
    m, n_per = x.shape
    n_global = n_per * N_DEV
    m_blk = m // N_BLK

    m_half = m // 2
    first_send_blk = N_BLK // 2 - 1

    def body(x_ref, inv_ref, comm_ref, send_sems, recv_sems):
        my = lax.axis_index("i")
        b = pl.program_id(0)

        def half_rdma(row, h, t):
            return pltpu.make_async_remote_copy(
                src_ref=comm_ref.at[pl.ds(row, 1), pl.ds(h * m_half, m_half)],
                dst_ref=comm_ref.at[pl.ds(row, 1), pl.ds(h * m_half, m_half)],
                send_sem=send_sems.at[t, h],
                recv_sem=recv_sems.at[row, h],
                device_id=(t,),
                device_id_type=pl.DeviceIdType.MESH,
            )

        @pl.when(b == 0)
        def _():
            barrier_sem = pltpu.get_barrier_semaphore()
            for t in range(N_DEV):
                @pl.when(my != t)
                def _(t=t):
                    pl.semaphore_signal(
                        barrier_sem, inc=1,
                        device_id=(t,),
                        device_id_type=pl.DeviceIdType.MESH,
                    )

        xb = x_ref[:, :]
        xb2 = xb * xb
        ones_n = jnp.ones((1, n_per), jnp.float32)
        ssb = lax.dot_general(
            ones_n, xb2,
            dimension_numbers=(((1,), (1,)), ((), ())),
            preferred_element_type=jnp.float32,
        )
        comm_ref[pl.ds(my, 1), pl.ds(b * m_blk, m_blk)] = ssb

        @pl.when(b == first_send_blk)
        def _():
            barrier_sem = pltpu.get_barrier_semaphore()
            pl.semaphore_wait(barrier_sem, N_DEV - 1)
            for t in range(N_DEV):
                @pl.when(my != t)
                def _(t=t):
                    half_rdma(my, 0, t).start()

        @pl.when(b == N_BLK - 1)
        def _():
            for t in range(N_DEV):
                @pl.when(my != t)
                def _(t=t):
                    half_rdma(my, 1, t).start()

            for s in range(N_DEV):
                @pl.when(my != s)
                def _(s=s):
                    half_rdma(s, 0, s).wait_recv()
                    half_rdma(s, 1, s).wait_recv()

            for t in range(N_DEV):
                @pl.when(my != t)
                def _(t=t):
                    half_rdma(my, 0, t).wait_send()
                    half_rdma(my, 1, t).wait_send()

            ones_d = jnp.ones((N_DEV, 1), jnp.float32)
            total_col = lax.dot_general(
                comm_ref[:, :], ones_d,
                dimension_numbers=(((0,), (0,)), ((), ())),
                preferred_element_type=jnp.float32,
            )
            inv_ref[:, :] = lax.rsqrt(total_col * (1.0 / n_global) + EPS)

    return pl.pallas_call(
        body,
        grid=(N_BLK,),
        out_shape=jax.ShapeDtypeStruct((m, 1), jnp.float32),
        in_specs=[
            pl.BlockSpec((m_blk, n_per), lambda b: (b, 0),
                         memory_space=pltpu.VMEM),
        ],
        out_specs=pl.BlockSpec((m, 1), lambda b: (0, 0),
                               memory_space=pltpu.VMEM),
        scratch_shapes=[
            pltpu.VMEM((N_DEV, m), jnp.float32),
            pltpu.SemaphoreType.DMA((N_DEV, 2)),
            pltpu.SemaphoreType.DMA((N_DEV, 2)),
        ],
        compiler_params=pltpu.CompilerParams(
            collective_id=0,
            dimension_semantics=("arbitrary",),
            vmem_limit_bytes=100 * 1024 * 1024,
        ),
    )(x)


def _scale(x, g2, inv):
    m, n_per = x.shape
    m_blk = m // N_BLK_B

    def body(x_ref, g_ref, inv_ref, out_ref):
        out_ref[:, :] = x_ref[:, :] * inv_ref[:, :] * g_ref[:, :]

    return pl.pallas_call(
        body,
        grid=(N_BLK_B,),
        out_shape=jax.ShapeDtypeStruct((m, n_per), jnp.float32),
        in_specs=[
            pl.BlockSpec((m_blk, n_per), lambda b: (b, 0),
                         memory_space=pltpu.VMEM),
            pl.BlockSpec((1, n_per), lambda b: (0, 0),
                         memory_space=pltpu.VMEM),
            pl.BlockSpec((m_blk, 1), lambda b: (b, 0),
                         memory_space=pltpu.VMEM),
        ],
        out_specs=pl.BlockSpec((m_blk, n_per), lambda b: (b, 0),
                               memory_space=pltpu.VMEM),
        compiler_params=pltpu.CompilerParams(
            dimension_semantics=("arbitrary",),
            vmem_limit_bytes=100 * 1024 * 1024,
        ),
    )(x, g2, inv)


def kernel(x, gamma):
    m, n_per = x.shape
    g2 = gamma.reshape(1, n_per)
    inv = _allreduce_inv(x)
    return _scale(x, g2, inv)


# device time: 31233 ns/iter; 1.2132x vs baseline; 1.2132x over previous
import jax
import jax.numpy as jnp
from jax import lax
from jax.experimental import pallas as pl
from jax.experimental.pallas import tpu as pltpu

N_DEV = 16
EPS = 1e-5
N_BLK = 4


def _allreduce_inv(x):
    m, n_per = x.shape
    n_global = n_per * N_DEV
    m_blk = m // N_BLK

    wave_off = (0, 3 * m // 4)
    wave_len = (3 * m // 4, m // 4)
    first_send_blk = N_BLK - 2

    def body(x_ref, inv_ref, comm_ref, send_sems, recv_sems):
        my = lax.axis_index("i")
        b = pl.program_id(0)

        def wave_rdma(row, w, t):
            return pltpu.make_async_remote_copy(
                src_ref=comm_ref.at[pl.ds(row, 1), pl.ds(wave_off[w], wave_len[w])],
                dst_ref=comm_ref.at[pl.ds(row, 1), pl.ds(wave_off[w], wave_len[w])],
                send_sem=send_sems.at[t, w],
                recv_sem=recv_sems.at[row, w],
                device_id=(t,),
                device_id_type=pl.DeviceIdType.MESH,
            )

        @pl.when(b == 0)
        def _():
            barrier_sem = pltpu.get_barrier_semaphore()
            for t in range(N_DEV):
                @pl.when(my != t)
                def _(t=t):
                    pl.semaphore_signal(
                        barrier_sem, inc=1,
                        device_id=(t,),
                        device_id_type=pl.DeviceIdType.MESH,
                    )

        xb = x_ref[:, :]
        xb2 = xb * xb
        ones_n = jnp.ones((1, n_per), jnp.float32)
        ssb = lax.dot_general(
            ones_n, xb2,
            dimension_numbers=(((1,), (1,)), ((), ())),
            preferred_element_type=jnp.float32,
        )
        comm_ref[pl.ds(my, 1), pl.ds(b * m_blk, m_blk)] = ssb

        @pl.when(b == first_send_blk)
        def _():
            barrier_sem = pltpu.get_barrier_semaphore()
            pl.semaphore_wait(barrier_sem, N_DEV - 1)
            for t in range(N_DEV):
                @pl.when(my != t)
                def _(t=t):
                    wave_rdma(my, 0, t).start()

        @pl.when(b == N_BLK - 1)
        def _():
            for t in range(N_DEV):
                @pl.when(my != t)
                def _(t=t):
                    wave_rdma(my, 1, t).start()

            for s in range(N_DEV):
                @pl.when(my != s)
                def _(s=s):
                    wave_rdma(s, 0, s).wait_recv()
                    wave_rdma(s, 1, s).wait_recv()

            for t in range(N_DEV):
                @pl.when(my != t)
                def _(t=t):
                    wave_rdma(my, 0, t).wait_send()
                    wave_rdma(my, 1, t).wait_send()

            ones_d = jnp.ones((N_DEV, 1), jnp.float32)
            total_col = lax.dot_general(
                comm_ref[:, :], ones_d,
                dimension_numbers=(((0,), (0,)), ((), ())),
                preferred_element_type=jnp.float32,
            )
            inv_ref[:, :] = lax.rsqrt(total_col * (1.0 / n_global) + EPS)

    return pl.pallas_call(
        body,
        grid=(N_BLK,),
        out_shape=jax.ShapeDtypeStruct((m, 1), jnp.float32),
        in_specs=[
            pl.BlockSpec((m_blk, n_per), lambda b: (b, 0),
                         memory_space=pltpu.VMEM),
        ],
        out_specs=pl.BlockSpec((m, 1), lambda b: (0, 0),
                               memory_space=pltpu.VMEM),
        scratch_shapes=[
            pltpu.VMEM((N_DEV, m), jnp.float32),
            pltpu.SemaphoreType.DMA((N_DEV, 2)),
            pltpu.SemaphoreType.DMA((N_DEV, 2)),
        ],
        compiler_params=pltpu.CompilerParams(
            collective_id=0,
            dimension_semantics=("arbitrary",),
            vmem_limit_bytes=100 * 1024 * 1024,
        ),
    )(x)


def _scale(x, g2, inv):
    m, n_per = x.shape

    def body(x_ref, g_ref, inv_ref, out_ref):
        out_ref[:, :] = x_ref[:, :] * inv_ref[:, :] * g_ref[:, :]

    return pl.pallas_call(
        body,
        out_shape=jax.ShapeDtypeStruct((m, n_per), jnp.float32),
        in_specs=[
            pl.BlockSpec(memory_space=pltpu.VMEM),
            pl.BlockSpec(memory_space=pltpu.VMEM),
            pl.BlockSpec(memory_space=pltpu.VMEM),
        ],
        out_specs=pl.BlockSpec(memory_space=pltpu.VMEM),
        compiler_params=pltpu.CompilerParams(
            vmem_limit_bytes=100 * 1024 * 1024,
        ),
    )(x, g2, inv)


def kernel(x, gamma):
    m, n_per = x.shape
    g2 = gamma.reshape(1, n_per)
    inv = _allreduce_inv(x)
    return _scale(x, g2, inv)


# device time: 30718 ns/iter; 1.2335x vs baseline; 1.0168x over previous
import jax
import jax.numpy as jnp
from jax import lax
from jax.experimental import pallas as pl
from jax.experimental.pallas import tpu as pltpu

N_DEV = 16
EPS = 1e-5
N_BLK = 4


def _allreduce_inv(x):
    m, n_per = x.shape
    n_global = n_per * N_DEV
    m_blk = m // N_BLK

    wave_off = (0, 3 * m // 4)
    wave_len = (3 * m // 4, m // 4)
    first_send_blk = N_BLK - 2

    def body(x_ref, inv_ref, comm_ref, send_sems, recv_sems):
        my = lax.axis_index("i")
        b = pl.program_id(0)

        def wave_rdma(row, w, t):
            return pltpu.make_async_remote_copy(
                src_ref=comm_ref.at[pl.ds(row, 1), pl.ds(wave_off[w], wave_len[w])],
                dst_ref=comm_ref.at[pl.ds(row, 1), pl.ds(wave_off[w], wave_len[w])],
                send_sem=send_sems.at[t, w],
                recv_sem=recv_sems.at[row, w],
                device_id=(t,),
                device_id_type=pl.DeviceIdType.MESH,
            )

        @pl.when(b == 0)
        def _():
            barrier_sem = pltpu.get_barrier_semaphore()
            for t in range(N_DEV):
                @pl.when(my != t)
                def _(t=t):
                    pl.semaphore_signal(
                        barrier_sem, inc=1,
                        device_id=(t,),
                        device_id_type=pl.DeviceIdType.MESH,
                    )

        xb = x_ref[:, :]
        xb2 = xb * xb
        ones_n = jnp.ones((1, n_per), jnp.float32)
        ssb = lax.dot_general(
            ones_n, xb2,
            dimension_numbers=(((1,), (1,)), ((), ())),
            preferred_element_type=jnp.float32,
        )
        comm_ref[pl.ds(my, 1), pl.ds(b * m_blk, m_blk)] = ssb

        @pl.when(b == first_send_blk)
        def _():
            barrier_sem = pltpu.get_barrier_semaphore()
            pl.semaphore_wait(barrier_sem, N_DEV - 1)
            for t in range(N_DEV):
                @pl.when(my != t)
                def _(t=t):
                    wave_rdma(my, 0, t).start()

        @pl.when(b == N_BLK - 1)
        def _():
            for t in range(N_DEV):
                @pl.when(my != t)
                def _(t=t):
                    wave_rdma(my, 1, t).start()

            ones_d = jnp.ones((N_DEV, 1), jnp.float32)

            def reduce_cols(lo, ln):
                total_col = lax.dot_general(
                    comm_ref[:, lo:lo + ln], ones_d,
                    dimension_numbers=(((0,), (0,)), ((), ())),
                    preferred_element_type=jnp.float32,
                )
                inv_ref[lo:lo + ln, :] = lax.rsqrt(
                    total_col * (1.0 / n_global) + EPS
                )

            for s in range(N_DEV):
                @pl.when(my != s)
                def _(s=s):
                    wave_rdma(s, 0, s).wait_recv()
            reduce_cols(wave_off[0], wave_len[0])

            for s in range(N_DEV):
                @pl.when(my != s)
                def _(s=s):
                    wave_rdma(s, 1, s).wait_recv()
            reduce_cols(wave_off[1], wave_len[1])

            for t in range(N_DEV):
                @pl.when(my != t)
                def _(t=t):
                    wave_rdma(my, 0, t).wait_send()
                    wave_rdma(my, 1, t).wait_send()

    return pl.pallas_call(
        body,
        grid=(N_BLK,),
        out_shape=jax.ShapeDtypeStruct((m, 1), jnp.float32),
        in_specs=[
            pl.BlockSpec((m_blk, n_per), lambda b: (b, 0),
                         memory_space=pltpu.VMEM),
        ],
        out_specs=pl.BlockSpec((m, 1), lambda b: (0, 0),
                               memory_space=pltpu.VMEM),
        scratch_shapes=[
            pltpu.VMEM((N_DEV, m), jnp.float32),
            pltpu.SemaphoreType.DMA((N_DEV, 2)),
            pltpu.SemaphoreType.DMA((N_DEV, 2)),
        ],
        compiler_params=pltpu.CompilerParams(
            collective_id=0,
            dimension_semantics=("arbitrary",),
            vmem_limit_bytes=100 * 1024 * 1024,
        ),
    )(x)


def _scale(x, g2, inv):
    m, n_per = x.shape

    def body(x_ref, g_ref, inv_ref, out_ref):
        out_ref[:, :] = x_ref[:, :] * inv_ref[:, :] * g_ref[:, :]

    return pl.pallas_call(
        body,
        out_shape=jax.ShapeDtypeStruct((m, n_per), jnp.float32),
        in_specs=[
            pl.BlockSpec(memory_space=pltpu.VMEM),
            pl.BlockSpec(memory_space=pltpu.VMEM),
            pl.BlockSpec(memory_space=pltpu.VMEM),
        ],
        out_specs=pl.BlockSpec(memory_space=pltpu.VMEM),
        compiler_params=pltpu.CompilerParams(
            vmem_limit_bytes=100 * 1024 * 1024,
        ),
    )(x, g2, inv)


def kernel(x, gamma):
    m, n_per = x.shape
    g2 = gamma.reshape(1, n_per)
    inv = _allreduce_inv(x)
    return _scale(x, g2, inv)
